# baseline (speedup 1.0000x reference)
import functools

import jax
import jax.numpy as jnp
from jax import lax
from jax.experimental import pallas as pl
from jax.experimental.pallas import tpu as pltpu
from jax.experimental.pallas import tpu_sc as plsc

NC = 2
NS = 16
LANES = 16


def _deg_call(npad, ept):
    @functools.partial(
        pl.kernel,
        out_type=jax.ShapeDtypeStruct((NC, NS, npad), jnp.float32),
        mesh=plsc.VectorSubcoreMesh(core_axis_name="c", subcore_axis_name="s"),
        compiler_params=pltpu.CompilerParams(needs_layout_passes=False),
        scratch_types=[
            pltpu.VMEM((ept,), jnp.int32),
            pltpu.VMEM((npad,), jnp.float32),
        ],
    )
    def deg_kernel(dst_hbm, zdeg_hbm, out_hbm, dstv, degl):
        c = lax.axis_index("c")
        s = lax.axis_index("s")
        pltpu.sync_copy(dst_hbm.at[c, s], dstv)
        pltpu.sync_copy(zdeg_hbm, degl)
        ones = jnp.ones((LANES,), jnp.float32)

        def body(i, carry):
            idx = dstv[pl.ds(i * LANES, LANES)]
            plsc.addupdate_scatter(degl, [idx], ones)
            return carry

        lax.fori_loop(0, ept // LANES, body, 0)
        pltpu.sync_copy(degl, out_hbm.at[c, s])

    return deg_kernel


def _prop_call(npad, hh, nchunks, chunk):
    rows_per_tile = npad // NS

    @functools.partial(
        pl.kernel,
        out_type=jax.ShapeDtypeStruct((NC, npad, hh), jnp.bfloat16),
        mesh=plsc.VectorSubcoreMesh(core_axis_name="c", subcore_axis_name="s"),
        compiler_params=pltpu.CompilerParams(use_tc_tiling_on_sc=False),
        scratch_types=[
            pltpu.VMEM((nchunks, chunk), jnp.int32),
            pltpu.VMEM((nchunks, chunk), jnp.int32),
            pltpu.VMEM((chunk, hh), jnp.bfloat16),
            pltpu.VMEM((chunk, hh), jnp.bfloat16),
            pltpu.VMEM_SHARED((npad, hh), jnp.bfloat16),
            pltpu.SemaphoreType.DMA,
            pltpu.SemaphoreType.DMA,
        ],
    )
    def prop_kernel(xs_hbm, src_hbm, dst_hbm, zrows_hbm, out_hbm,
                    srcv, dstv, rows_a, rows_b, accs, sem_a, sem_b):
        c = lax.axis_index("c")
        s = lax.axis_index("s")
        pltpu.sync_copy(src_hbm.at[c, s], srcv)
        pltpu.sync_copy(dst_hbm.at[s], dstv)
        sl = pl.ds(s * rows_per_tile, rows_per_tile)
        pltpu.sync_copy(zrows_hbm, accs.at[sl])
        plsc.subcore_barrier()

        pltpu.async_copy(xs_hbm.at[srcv.at[0]], rows_a, sem_a)

        def body(j, carry):
            @pl.when(j % 2 == 0)
            def _even():
                pltpu.async_copy(xs_hbm.at[srcv.at[j + 1]], rows_b, sem_b)
                pltpu.make_async_copy(xs_hbm.at[srcv.at[j]], rows_a,
                                      sem_a).wait()
                pltpu.sync_copy(rows_a, accs.at[dstv.at[j]], add=True)

            @pl.when(j % 2 == 1)
            def _odd():
                pltpu.async_copy(xs_hbm.at[srcv.at[j + 1]], rows_a, sem_a)
                pltpu.make_async_copy(xs_hbm.at[srcv.at[j]], rows_b,
                                      sem_b).wait()
                pltpu.sync_copy(rows_b, accs.at[dstv.at[j]], add=True)

            return carry

        lax.fori_loop(0, nchunks - 1, body, 0)
        last = nchunks - 1
        if last % 2 == 0:
            pltpu.make_async_copy(xs_hbm.at[srcv.at[last]], rows_a,
                                  sem_a).wait()
            pltpu.sync_copy(rows_a, accs.at[dstv.at[last]], add=True)
        else:
            pltpu.make_async_copy(xs_hbm.at[srcv.at[last]], rows_b,
                                  sem_b).wait()
            pltpu.sync_copy(rows_b, accs.at[dstv.at[last]], add=True)

        plsc.subcore_barrier()
        pltpu.sync_copy(accs.at[sl], out_hbm.at[c, sl])

    return prop_kernel


def _tc1_body(x_ref, w_ref, deg_ref, xs_ref, dinv_ref):
    n = x_ref.shape[0]
    npad = deg_ref.shape[1]
    ones = jnp.ones((deg_ref.shape[0], 1), jnp.float32)
    deg_col = lax.dot_general(deg_ref[...], ones, (((0,), (0,)), ((), ())),
                              preferred_element_type=jnp.float32)
    dinv = lax.rsqrt(deg_col + 1.0)
    xw = jnp.dot(x_ref[...], w_ref[...], preferred_element_type=jnp.float32)
    xs = (xw * dinv[:n]).astype(jnp.bfloat16)
    hh = xs.shape[1] // 2
    zpad = jnp.zeros((npad - n, hh), jnp.bfloat16)
    xs_ref[pl.ds(0, n)] = xs[:, :hh]
    xs_ref[pl.ds(n, npad - n)] = zpad
    xs_ref[pl.ds(npad, n)] = xs[:, hh:]
    xs_ref[pl.ds(npad + n, npad - n)] = zpad
    dinv_ref[...] = dinv


def _tc2_body(acc_ref, xs_ref, dinv_ref, w2a_ref, w2b_ref, b1_ref, out_ref):
    dinv = dinv_ref[...]
    hh = acc_ref.shape[2]
    npad = acc_ref.shape[1]
    xsa = xs_ref[pl.ds(0, npad)].astype(jnp.float32)
    xsb = xs_ref[pl.ds(npad, npad)].astype(jnp.float32)
    s0 = acc_ref[0].astype(jnp.float32) + xsa
    s1 = acc_ref[1].astype(jnp.float32) + xsb
    h1a = jnp.maximum(dinv * s0 + b1_ref[:, :hh], 0.0)
    h1b = jnp.maximum(dinv * s1 + b1_ref[:, hh:], 0.0)
    xw2 = (jnp.dot(h1a, w2a_ref[...], preferred_element_type=jnp.float32)
           + jnp.dot(h1b, w2b_ref[...], preferred_element_type=jnp.float32))
    xs2 = (xw2 * dinv).astype(jnp.bfloat16)
    out_ref[pl.ds(0, npad)] = xs2[:, :hh]
    out_ref[pl.ds(npad, npad)] = xs2[:, hh:]


def _tc3_body(num_groups, acc_ref, xs_ref, dinv_ref, b2_ref, batch_ref,
              wpa_ref, wpb_ref, bp_ref, out_ref):
    dinv = dinv_ref[...]
    hh = acc_ref.shape[2]
    npad = acc_ref.shape[1]
    xsa = xs_ref[pl.ds(0, npad)].astype(jnp.float32)
    xsb = xs_ref[pl.ds(npad, npad)].astype(jnp.float32)
    s0 = acc_ref[0].astype(jnp.float32) + xsa
    s1 = acc_ref[1].astype(jnp.float32) + xsb
    h2a = jnp.maximum(dinv * s0 + b2_ref[:, :hh], 0.0)
    h2b = jnp.maximum(dinv * s1 + b2_ref[:, hh:], 0.0)
    g = lax.broadcasted_iota(jnp.int32, (1, num_groups), 1)
    onehot = (batch_ref[...] == g).astype(jnp.float32)
    dims = (((0,), (0,)), ((), ()))
    sums_a = lax.dot_general(onehot, h2a, dims,
                             preferred_element_type=jnp.float32)
    sums_b = lax.dot_general(onehot, h2b, dims,
                             preferred_element_type=jnp.float32)
    ones = jnp.ones((h2a.shape[0], 1), jnp.float32)
    counts = lax.dot_general(onehot, ones, dims,
                             preferred_element_type=jnp.float32)
    inv_counts = 1.0 / jnp.maximum(counts, 1.0)
    pa = sums_a * inv_counts
    pb = sums_b * inv_counts
    out_ref[...] = (jnp.dot(pa, wpa_ref[...],
                            preferred_element_type=jnp.float32)
                    + jnp.dot(pb, wpb_ref[...],
                              preferred_element_type=jnp.float32)
                    + bp_ref[...])


def kernel(x, edge_index, batch, W1, b1, W2, b2, Wp, bp):
    n, d = x.shape
    e = edge_index.shape[1]
    h = W1.shape[1]
    hh = h // 2
    out_dim = Wp.shape[1]
    num_groups = 64

    chunk = 128
    npad = ((n + 2047) // 2048) * 2048
    ept_d = ((e + NC * NS * LANES - 1) // (NC * NS * LANES)) * LANES
    ept_p = ((e + NS * chunk - 1) // (NS * chunk)) * chunk
    nchunks = ept_p // chunk

    src = edge_index[0]
    dst = edge_index[1]
    pad_d = jnp.full((NC * NS * ept_d - e,), n, jnp.int32)
    dstf = jnp.concatenate([dst, pad_d]).reshape(NC, NS, ept_d)
    pad_p = jnp.full((NS * ept_p - e,), n, jnp.int32)
    srcp1 = jnp.concatenate([src, pad_p]).reshape(NS, nchunks, chunk)
    srcp = jnp.stack([srcp1, srcp1 + npad])
    dstp = jnp.concatenate([dst, pad_p]).reshape(NS, nchunks, chunk)
    batch_col = jnp.concatenate(
        [batch, jnp.full((npad - n,), num_groups, jnp.int32)]).reshape(npad, 1)
    zdeg = jnp.zeros((npad,), jnp.float32)
    zrows = jnp.zeros((npad // NS, hh), jnp.bfloat16)

    deg32 = _deg_call(npad, ept_d)(dstf, zdeg).reshape(NC * NS, npad)

    xs1, dinv = pl.pallas_call(
        _tc1_body,
        out_shape=[
            jax.ShapeDtypeStruct((NC * npad, hh), jnp.bfloat16),
            jax.ShapeDtypeStruct((npad, 1), jnp.float32),
        ],
    )(x, W1, deg32)

    prop = _prop_call(npad, hh, nchunks, chunk)
    acc1 = prop(xs1, srcp, dstp, zrows)

    xs2 = pl.pallas_call(
        _tc2_body,
        out_shape=jax.ShapeDtypeStruct((NC * npad, hh), jnp.bfloat16),
    )(acc1, xs1, dinv, W2[:hh], W2[hh:], b1.reshape(1, h))

    acc2 = prop(xs2, srcp, dstp, zrows)

    out = pl.pallas_call(
        functools.partial(_tc3_body, num_groups),
        out_shape=jax.ShapeDtypeStruct((num_groups, out_dim), jnp.float32),
    )(acc2, xs2, dinv, b2.reshape(1, h), batch_col, Wp[:hh], Wp[hh:],
      bp.reshape(1, out_dim))
    return out


# speedup vs baseline: 1.0099x; 1.0099x over previous
import functools

import jax
import jax.numpy as jnp
from jax import lax
from jax.experimental import pallas as pl
from jax.experimental.pallas import tpu as pltpu
from jax.experimental.pallas import tpu_sc as plsc

NC = 2
NS = 16
LANES = 16


def _deg_call(npad, nchunks, chunk):
    half = (nchunks + 1) // 2
    k = chunk // LANES

    @functools.partial(
        pl.kernel,
        out_type=jax.ShapeDtypeStruct((NC, NS, npad), jnp.float32),
        mesh=plsc.VectorSubcoreMesh(core_axis_name="c", subcore_axis_name="s"),
        compiler_params=pltpu.CompilerParams(needs_layout_passes=False),
        scratch_types=[
            pltpu.VMEM((nchunks, chunk), jnp.int32),
            pltpu.VMEM((npad,), jnp.float32),
        ],
    )
    def deg_kernel(edges_hbm, zdeg_hbm, out_hbm, dstv, degl):
        c = lax.axis_index("c")
        s = lax.axis_index("s")
        pltpu.sync_copy(edges_hbm.at[1, s], dstv)
        pltpu.sync_copy(zdeg_hbm, degl)
        ones = jnp.ones((LANES,), jnp.float32)

        def body(i, carry):
            j = i // k
            t = i % k
            idx = dstv[j, pl.ds(t * LANES, LANES)]
            plsc.addupdate_scatter(degl, [idx], ones)
            return carry

        lax.fori_loop(c * half * k, (half + c * (nchunks - half)) * k,
                      body, 0)
        pltpu.sync_copy(degl, out_hbm.at[c, s])

    return deg_kernel


def _prop_call(npad, hh, nchunks, chunk):
    rows_per_tile = npad // NS

    @functools.partial(
        pl.kernel,
        out_type=jax.ShapeDtypeStruct((NC, npad, hh), jnp.bfloat16),
        mesh=plsc.VectorSubcoreMesh(core_axis_name="c", subcore_axis_name="s"),
        compiler_params=pltpu.CompilerParams(use_tc_tiling_on_sc=False),
        scratch_types=[
            pltpu.VMEM((nchunks, chunk), jnp.int32),
            pltpu.VMEM((nchunks, chunk), jnp.int32),
            pltpu.VMEM((chunk, hh), jnp.bfloat16),
            pltpu.VMEM((chunk, hh), jnp.bfloat16),
            pltpu.VMEM_SHARED((npad, hh), jnp.bfloat16),
            pltpu.SemaphoreType.DMA,
            pltpu.SemaphoreType.DMA,
        ],
    )
    def prop_kernel(xsa_hbm, xsb_hbm, edges_hbm, zrows_hbm, out_hbm,
                    srcv, dstv, rows_a, rows_b, accs, sem_a, sem_b):
        c = lax.axis_index("c")
        s = lax.axis_index("s")
        pltpu.sync_copy(edges_hbm.at[0, s], srcv)
        pltpu.sync_copy(edges_hbm.at[1, s], dstv)
        sl = pl.ds(s * rows_per_tile, rows_per_tile)
        pltpu.sync_copy(zrows_hbm, accs.at[sl])
        plsc.subcore_barrier()

        def run_edges(xs_hbm):
            pltpu.async_copy(xs_hbm.at[srcv.at[0]], rows_a, sem_a)

            def body(j, carry):
                @pl.when(j % 2 == 0)
                def _even():
                    pltpu.async_copy(xs_hbm.at[srcv.at[j + 1]], rows_b, sem_b)
                    pltpu.make_async_copy(xs_hbm.at[srcv.at[j]], rows_a,
                                          sem_a).wait()
                    pltpu.sync_copy(rows_a, accs.at[dstv.at[j]], add=True)

                @pl.when(j % 2 == 1)
                def _odd():
                    pltpu.async_copy(xs_hbm.at[srcv.at[j + 1]], rows_a, sem_a)
                    pltpu.make_async_copy(xs_hbm.at[srcv.at[j]], rows_b,
                                          sem_b).wait()
                    pltpu.sync_copy(rows_b, accs.at[dstv.at[j]], add=True)

                return carry

            lax.fori_loop(0, nchunks - 1, body, 0)
            last = nchunks - 1
            if last % 2 == 0:
                pltpu.make_async_copy(xs_hbm.at[srcv.at[last]], rows_a,
                                      sem_a).wait()
                pltpu.sync_copy(rows_a, accs.at[dstv.at[last]], add=True)
            else:
                pltpu.make_async_copy(xs_hbm.at[srcv.at[last]], rows_b,
                                      sem_b).wait()
                pltpu.sync_copy(rows_b, accs.at[dstv.at[last]], add=True)

        @pl.when(c == 0)
        def _core0():
            run_edges(xsa_hbm)

        @pl.when(c == 1)
        def _core1():
            run_edges(xsb_hbm)

        plsc.subcore_barrier()
        pltpu.sync_copy(accs.at[sl], out_hbm.at[c, sl])

    return prop_kernel


def _tc1_body(x_ref, w_ref, deg_ref, xsa_ref, xsb_ref, dinv_ref):
    n = x_ref.shape[0]
    npad = deg_ref.shape[1]
    ones = jnp.ones((deg_ref.shape[0], 1), jnp.float32)
    deg_col = lax.dot_general(deg_ref[...], ones, (((0,), (0,)), ((), ())),
                              preferred_element_type=jnp.float32)
    dinv = lax.rsqrt(deg_col + 1.0)
    xw = jnp.dot(x_ref[...], w_ref[...], preferred_element_type=jnp.float32)
    xs = (xw * dinv[:n]).astype(jnp.bfloat16)
    hh = xs.shape[1] // 2
    zpad = jnp.zeros((npad - n, hh), jnp.bfloat16)
    xsa_ref[pl.ds(0, n)] = xs[:, :hh]
    xsa_ref[pl.ds(n, npad - n)] = zpad
    xsb_ref[pl.ds(0, n)] = xs[:, hh:]
    xsb_ref[pl.ds(n, npad - n)] = zpad
    dinv_ref[...] = dinv


def _tc2_body(acc_ref, xsa_ref, xsb_ref, dinv_ref, w2a_ref, w2b_ref, b1_ref,
              outa_ref, outb_ref):
    dinv = dinv_ref[...]
    hh = acc_ref.shape[2]
    s0 = acc_ref[0].astype(jnp.float32) + xsa_ref[...].astype(jnp.float32)
    s1 = acc_ref[1].astype(jnp.float32) + xsb_ref[...].astype(jnp.float32)
    h1a = jnp.maximum(dinv * s0 + b1_ref[:, :hh], 0.0)
    h1b = jnp.maximum(dinv * s1 + b1_ref[:, hh:], 0.0)
    xw2 = (jnp.dot(h1a, w2a_ref[...], preferred_element_type=jnp.float32)
           + jnp.dot(h1b, w2b_ref[...], preferred_element_type=jnp.float32))
    xs2 = (xw2 * dinv).astype(jnp.bfloat16)
    outa_ref[...] = xs2[:, :hh]
    outb_ref[...] = xs2[:, hh:]


def _tc3_body(num_groups, acc_ref, xsa_ref, xsb_ref, dinv_ref, b2_ref,
              batch_ref, wpa_ref, wpb_ref, bp_ref, out_ref):
    dinv = dinv_ref[...]
    hh = acc_ref.shape[2]
    s0 = acc_ref[0].astype(jnp.float32) + xsa_ref[...].astype(jnp.float32)
    s1 = acc_ref[1].astype(jnp.float32) + xsb_ref[...].astype(jnp.float32)
    h2a = jnp.maximum(dinv * s0 + b2_ref[:, :hh], 0.0)
    h2b = jnp.maximum(dinv * s1 + b2_ref[:, hh:], 0.0)
    g = lax.broadcasted_iota(jnp.int32, (1, num_groups), 1)
    onehot = (batch_ref[...] == g).astype(jnp.float32)
    dims = (((0,), (0,)), ((), ()))
    sums_a = lax.dot_general(onehot, h2a, dims,
                             preferred_element_type=jnp.float32)
    sums_b = lax.dot_general(onehot, h2b, dims,
                             preferred_element_type=jnp.float32)
    ones = jnp.ones((h2a.shape[0], 1), jnp.float32)
    counts = lax.dot_general(onehot, ones, dims,
                             preferred_element_type=jnp.float32)
    inv_counts = 1.0 / jnp.maximum(counts, 1.0)
    pa = sums_a * inv_counts
    pb = sums_b * inv_counts
    out_ref[...] = (jnp.dot(pa, wpa_ref[...],
                            preferred_element_type=jnp.float32)
                    + jnp.dot(pb, wpb_ref[...],
                              preferred_element_type=jnp.float32)
                    + bp_ref[...])


def kernel(x, edge_index, batch, W1, b1, W2, b2, Wp, bp):
    n, d = x.shape
    e = edge_index.shape[1]
    h = W1.shape[1]
    hh = h // 2
    out_dim = Wp.shape[1]
    num_groups = 64

    chunk = 128
    npad = ((n + 2047) // 2048) * 2048
    ept = ((e + NS * chunk - 1) // (NS * chunk)) * chunk
    nchunks = ept // chunk

    edges4 = jnp.pad(edge_index, ((0, 0), (0, NS * ept - e)),
                     constant_values=n).reshape(2, NS, nchunks, chunk)
    batch_col = jnp.concatenate(
        [batch, jnp.full((npad - n,), num_groups, jnp.int32)]).reshape(npad, 1)
    zdeg = jnp.zeros((npad,), jnp.float32)
    zrows = jnp.zeros((npad // NS, hh), jnp.bfloat16)

    deg32 = _deg_call(npad, nchunks, chunk)(edges4, zdeg).reshape(
        NC * NS, npad)

    xs1a, xs1b, dinv = pl.pallas_call(
        _tc1_body,
        out_shape=[
            jax.ShapeDtypeStruct((npad, hh), jnp.bfloat16),
            jax.ShapeDtypeStruct((npad, hh), jnp.bfloat16),
            jax.ShapeDtypeStruct((npad, 1), jnp.float32),
        ],
    )(x, W1, deg32)

    prop = _prop_call(npad, hh, nchunks, chunk)
    acc1 = prop(xs1a, xs1b, edges4, zrows)

    xs2a, xs2b = pl.pallas_call(
        _tc2_body,
        out_shape=[
            jax.ShapeDtypeStruct((npad, hh), jnp.bfloat16),
            jax.ShapeDtypeStruct((npad, hh), jnp.bfloat16),
        ],
    )(acc1, xs1a, xs1b, dinv, W2[:hh], W2[hh:], b1.reshape(1, h))

    acc2 = prop(xs2a, xs2b, edges4, zrows)

    out = pl.pallas_call(
        functools.partial(_tc3_body, num_groups),
        out_shape=jax.ShapeDtypeStruct((num_groups, out_dim), jnp.float32),
    )(acc2, xs2a, xs2b, dinv, b2.reshape(1, h), batch_col, Wp[:hh], Wp[hh:],
      bp.reshape(1, out_dim))
    return out


# speedup vs baseline: 1.0873x; 1.0767x over previous
import functools

import jax
import jax.numpy as jnp
from jax import lax
from jax.experimental import pallas as pl
from jax.experimental.pallas import tpu as pltpu
from jax.experimental.pallas import tpu_sc as plsc

NC = 2
NS = 16
LANES = 16


def _deg_call(npad, nchunks, chunk):
    half = (nchunks + 1) // 2
    k = chunk // LANES

    @functools.partial(
        pl.kernel,
        out_type=jax.ShapeDtypeStruct((NC, NS, npad), jnp.float32),
        mesh=plsc.VectorSubcoreMesh(core_axis_name="c", subcore_axis_name="s"),
        compiler_params=pltpu.CompilerParams(needs_layout_passes=False),
        scratch_types=[
            pltpu.VMEM((nchunks, chunk), jnp.int32),
            pltpu.VMEM((npad,), jnp.float32),
        ],
    )
    def deg_kernel(edges_hbm, zdeg_hbm, out_hbm, dstv, degl):
        c = lax.axis_index("c")
        s = lax.axis_index("s")
        pltpu.sync_copy(edges_hbm.at[1, s], dstv)
        pltpu.sync_copy(zdeg_hbm, degl)
        ones = jnp.ones((LANES,), jnp.float32)

        def body(i, carry):
            j = i // k
            t = i % k
            idx = dstv[j, pl.ds(t * LANES, LANES)]
            plsc.addupdate_scatter(degl, [idx], ones)
            return carry

        lax.fori_loop(c * half * k, (half + c * (nchunks - half)) * k,
                      body, 0)
        pltpu.sync_copy(degl, out_hbm.at[c, s])

    return deg_kernel


def _prop_call(npad, hh, nchunks, chunk):
    rows_per_tile = npad // NS

    @functools.partial(
        pl.kernel,
        out_type=jax.ShapeDtypeStruct((NC, npad, hh), jnp.bfloat16),
        mesh=plsc.VectorSubcoreMesh(core_axis_name="c", subcore_axis_name="s"),
        compiler_params=pltpu.CompilerParams(use_tc_tiling_on_sc=False),
        scratch_types=[
            pltpu.VMEM((nchunks, chunk), jnp.int32),
            pltpu.VMEM((nchunks, chunk), jnp.int32),
            pltpu.VMEM((4, chunk, hh), jnp.bfloat16),
            pltpu.VMEM_SHARED((npad, hh), jnp.bfloat16),
            pltpu.SemaphoreType.DMA((4,)),
            pltpu.SemaphoreType.DMA((4,)),
        ],
    )
    def prop_kernel(xsa_hbm, xsb_hbm, edges_hbm, zrows_hbm, out_hbm,
                    srcv, dstv, rows, accs, gsem, ssem):
        c = lax.axis_index("c")
        s = lax.axis_index("s")
        pltpu.sync_copy(edges_hbm.at[0, s], srcv)
        pltpu.sync_copy(edges_hbm.at[1, s], dstv)
        sl = pl.ds(s * rows_per_tile, rows_per_tile)
        pltpu.sync_copy(zrows_hbm, accs.at[sl])
        plsc.subcore_barrier()

        def run_edges(xs_hbm):
            def gather(j, b):
                pltpu.async_copy(xs_hbm.at[srcv.at[j]], rows.at[b],
                                 gsem.at[b])

            def wait_gather(j, b):
                pltpu.make_async_copy(xs_hbm.at[srcv.at[j]], rows.at[b],
                                      gsem.at[b]).wait()

            def scatter(j, b):
                pltpu.async_copy(rows.at[b], accs.at[dstv.at[j]],
                                 ssem.at[b], add=True)

            def wait_scatter(j, b):
                pltpu.make_async_copy(rows.at[b], accs.at[dstv.at[j]],
                                      ssem.at[b]).wait()

            gather(0, 0)
            gather(1, 1)
            wait_gather(0, 0)
            scatter(0, 0)
            gather(2, 2)
            wait_gather(1, 1)
            scatter(1, 1)
            gather(3, 3)

            def body(j, carry):
                for b in range(4):
                    @pl.when(j % 4 == b)
                    def _slot(b=b):
                        b2 = (b + 2) % 4
                        wait_gather(j, b)
                        scatter(j, b)
                        wait_scatter(j - 2, b2)
                        gather(j + 2, b2)
                return carry

            lax.fori_loop(2, nchunks - 2, body, 0)
            for j in (nchunks - 2, nchunks - 1):
                b = j % 4
                wait_gather(j, b)
                scatter(j, b)
            for j in range(nchunks - 4, nchunks):
                wait_scatter(j, j % 4)

        @pl.when(c == 0)
        def _core0():
            run_edges(xsa_hbm)

        @pl.when(c == 1)
        def _core1():
            run_edges(xsb_hbm)

        plsc.subcore_barrier()
        pltpu.sync_copy(accs.at[sl], out_hbm.at[c, sl])

    return prop_kernel


def _tc1_body(x_ref, w_ref, deg_ref, xsa_ref, xsb_ref, dinv_ref):
    n = x_ref.shape[0]
    npad = deg_ref.shape[1]
    ones = jnp.ones((deg_ref.shape[0], 1), jnp.float32)
    deg_col = lax.dot_general(deg_ref[...], ones, (((0,), (0,)), ((), ())),
                              preferred_element_type=jnp.float32)
    dinv = lax.rsqrt(deg_col + 1.0)
    xw = jnp.dot(x_ref[...], w_ref[...], preferred_element_type=jnp.float32)
    xs = (xw * dinv[:n]).astype(jnp.bfloat16)
    hh = xs.shape[1] // 2
    zpad = jnp.zeros((npad - n, hh), jnp.bfloat16)
    xsa_ref[pl.ds(0, n)] = xs[:, :hh]
    xsa_ref[pl.ds(n, npad - n)] = zpad
    xsb_ref[pl.ds(0, n)] = xs[:, hh:]
    xsb_ref[pl.ds(n, npad - n)] = zpad
    dinv_ref[...] = dinv


def _tc2_body(acc_ref, xsa_ref, xsb_ref, dinv_ref, w2a_ref, w2b_ref, b1_ref,
              outa_ref, outb_ref):
    dinv = dinv_ref[...]
    hh = acc_ref.shape[2]
    s0 = acc_ref[0].astype(jnp.float32) + xsa_ref[...].astype(jnp.float32)
    s1 = acc_ref[1].astype(jnp.float32) + xsb_ref[...].astype(jnp.float32)
    h1a = jnp.maximum(dinv * s0 + b1_ref[:, :hh], 0.0)
    h1b = jnp.maximum(dinv * s1 + b1_ref[:, hh:], 0.0)
    xw2 = (jnp.dot(h1a, w2a_ref[...], preferred_element_type=jnp.float32)
           + jnp.dot(h1b, w2b_ref[...], preferred_element_type=jnp.float32))
    xs2 = (xw2 * dinv).astype(jnp.bfloat16)
    outa_ref[...] = xs2[:, :hh]
    outb_ref[...] = xs2[:, hh:]


def _tc3_body(num_groups, acc_ref, xsa_ref, xsb_ref, dinv_ref, b2_ref,
              batch_ref, wpa_ref, wpb_ref, bp_ref, out_ref):
    dinv = dinv_ref[...]
    hh = acc_ref.shape[2]
    s0 = acc_ref[0].astype(jnp.float32) + xsa_ref[...].astype(jnp.float32)
    s1 = acc_ref[1].astype(jnp.float32) + xsb_ref[...].astype(jnp.float32)
    h2a = jnp.maximum(dinv * s0 + b2_ref[:, :hh], 0.0)
    h2b = jnp.maximum(dinv * s1 + b2_ref[:, hh:], 0.0)
    g = lax.broadcasted_iota(jnp.int32, (1, num_groups), 1)
    onehot = (batch_ref[...] == g).astype(jnp.float32)
    dims = (((0,), (0,)), ((), ()))
    sums_a = lax.dot_general(onehot, h2a, dims,
                             preferred_element_type=jnp.float32)
    sums_b = lax.dot_general(onehot, h2b, dims,
                             preferred_element_type=jnp.float32)
    ones = jnp.ones((h2a.shape[0], 1), jnp.float32)
    counts = lax.dot_general(onehot, ones, dims,
                             preferred_element_type=jnp.float32)
    inv_counts = 1.0 / jnp.maximum(counts, 1.0)
    pa = sums_a * inv_counts
    pb = sums_b * inv_counts
    out_ref[...] = (jnp.dot(pa, wpa_ref[...],
                            preferred_element_type=jnp.float32)
                    + jnp.dot(pb, wpb_ref[...],
                              preferred_element_type=jnp.float32)
                    + bp_ref[...])


def kernel(x, edge_index, batch, W1, b1, W2, b2, Wp, bp):
    n, d = x.shape
    e = edge_index.shape[1]
    h = W1.shape[1]
    hh = h // 2
    out_dim = Wp.shape[1]
    num_groups = 64

    chunk = 128
    npad = ((n + 2047) // 2048) * 2048
    ept = ((e + NS * chunk - 1) // (NS * chunk)) * chunk
    nchunks = ept // chunk

    edges4 = jnp.pad(edge_index, ((0, 0), (0, NS * ept - e)),
                     constant_values=n).reshape(2, NS, nchunks, chunk)
    batch_col = jnp.concatenate(
        [batch, jnp.full((npad - n,), num_groups, jnp.int32)]).reshape(npad, 1)
    zdeg = jnp.zeros((npad,), jnp.float32)
    zrows = jnp.zeros((npad // NS, hh), jnp.bfloat16)

    deg32 = _deg_call(npad, nchunks, chunk)(edges4, zdeg).reshape(
        NC * NS, npad)

    xs1a, xs1b, dinv = pl.pallas_call(
        _tc1_body,
        out_shape=[
            jax.ShapeDtypeStruct((npad, hh), jnp.bfloat16),
            jax.ShapeDtypeStruct((npad, hh), jnp.bfloat16),
            jax.ShapeDtypeStruct((npad, 1), jnp.float32),
        ],
    )(x, W1, deg32)

    prop = _prop_call(npad, hh, nchunks, chunk)
    acc1 = prop(xs1a, xs1b, edges4, zrows)

    xs2a, xs2b = pl.pallas_call(
        _tc2_body,
        out_shape=[
            jax.ShapeDtypeStruct((npad, hh), jnp.bfloat16),
            jax.ShapeDtypeStruct((npad, hh), jnp.bfloat16),
        ],
    )(acc1, xs1a, xs1b, dinv, W2[:hh], W2[hh:], b1.reshape(1, h))

    acc2 = prop(xs2a, xs2b, edges4, zrows)

    out = pl.pallas_call(
        functools.partial(_tc3_body, num_groups),
        out_shape=jax.ShapeDtypeStruct((num_groups, out_dim), jnp.float32),
    )(acc2, xs2a, xs2b, dinv, b2.reshape(1, h), batch_col, Wp[:hh], Wp[hh:],
      bp.reshape(1, out_dim))
    return out


# speedup vs baseline: 1.2230x; 1.1247x over previous
import functools

import jax
import jax.numpy as jnp
from jax import lax
from jax.experimental import pallas as pl
from jax.experimental.pallas import tpu as pltpu
from jax.experimental.pallas import tpu_sc as plsc

NC = 2
NS = 16
LANES = 16
SLOTS = 8
LOOKAHEAD = 4


def _deg_call(npad, nchunks, chunk):
    half = (nchunks + 1) // 2
    k = chunk // LANES

    @functools.partial(
        pl.kernel,
        out_type=jax.ShapeDtypeStruct((NC, NS, npad), jnp.float32),
        mesh=plsc.VectorSubcoreMesh(core_axis_name="c", subcore_axis_name="s"),
        compiler_params=pltpu.CompilerParams(needs_layout_passes=False),
        scratch_types=[
            pltpu.VMEM((nchunks, chunk), jnp.int32),
            pltpu.VMEM((npad,), jnp.float32),
        ],
    )
    def deg_kernel(edges_hbm, zdeg_hbm, out_hbm, dstv, degl):
        c = lax.axis_index("c")
        s = lax.axis_index("s")
        pltpu.sync_copy(edges_hbm.at[1, s], dstv)
        pltpu.sync_copy(zdeg_hbm, degl)
        ones = jnp.ones((LANES,), jnp.float32)

        def body(i, carry):
            j = i // k
            t = i % k
            idx = dstv[j, pl.ds(t * LANES, LANES)]
            plsc.addupdate_scatter(degl, [idx], ones)
            return carry

        lax.fori_loop(c * half * k, (half + c * (nchunks - half)) * k,
                      body, 0)
        pltpu.sync_copy(degl, out_hbm.at[c, s])

    return deg_kernel


def _prop_call(npad, hh, nchunks, chunk):
    rows_per_tile = npad // NS

    @functools.partial(
        pl.kernel,
        out_type=jax.ShapeDtypeStruct((NC, npad, hh), jnp.bfloat16),
        mesh=plsc.VectorSubcoreMesh(core_axis_name="c", subcore_axis_name="s"),
        compiler_params=pltpu.CompilerParams(use_tc_tiling_on_sc=False),
        scratch_types=[
            pltpu.VMEM((nchunks, chunk), jnp.int32),
            pltpu.VMEM((nchunks, chunk), jnp.int32),
            pltpu.VMEM((SLOTS, chunk, hh), jnp.bfloat16),
            pltpu.VMEM_SHARED((npad, hh), jnp.bfloat16),
            pltpu.SemaphoreType.DMA((SLOTS,)),
            pltpu.SemaphoreType.DMA((SLOTS,)),
        ],
    )
    def prop_kernel(xsa_hbm, xsb_hbm, edges_hbm, zrows_hbm, out_hbm,
                    srcv, dstv, rows, accs, gsem, ssem):
        c = lax.axis_index("c")
        s = lax.axis_index("s")
        pltpu.sync_copy(edges_hbm.at[0, s], srcv)
        pltpu.sync_copy(edges_hbm.at[1, s], dstv)
        sl = pl.ds(s * rows_per_tile, rows_per_tile)
        pltpu.sync_copy(zrows_hbm, accs.at[sl])
        plsc.subcore_barrier()

        def run_edges(xs_hbm):
            def gather(j, b):
                pltpu.async_copy(xs_hbm.at[srcv.at[j]], rows.at[b],
                                 gsem.at[b])

            def wait_gather(j, b):
                pltpu.make_async_copy(xs_hbm.at[srcv.at[j]], rows.at[b],
                                      gsem.at[b]).wait()

            def scatter(j, b):
                pltpu.async_copy(rows.at[b], accs.at[dstv.at[j]],
                                 ssem.at[b], add=True)

            def wait_scatter(j, b):
                pltpu.make_async_copy(rows.at[b], accs.at[dstv.at[j]],
                                      ssem.at[b]).wait()

            for j in range(LOOKAHEAD):
                gather(j, j)
            for j in range(LOOKAHEAD):
                wait_gather(j, j)
                scatter(j, j)
                gather(j + LOOKAHEAD, j + LOOKAHEAD)

            def body(j, carry):
                for b in range(SLOTS):
                    @pl.when(j % SLOTS == b)
                    def _slot(b=b):
                        b2 = (b + LOOKAHEAD) % SLOTS
                        wait_gather(j, b)
                        scatter(j, b)
                        wait_scatter(j - LOOKAHEAD, b2)
                        gather(j + LOOKAHEAD, b2)
                return carry

            lax.fori_loop(LOOKAHEAD, nchunks - LOOKAHEAD, body, 0)
            for j in range(nchunks - LOOKAHEAD, nchunks):
                b = j % SLOTS
                wait_gather(j, b)
                scatter(j, b)
            for j in range(nchunks - SLOTS, nchunks):
                wait_scatter(j, j % SLOTS)

        @pl.when(c == 0)
        def _core0():
            run_edges(xsa_hbm)

        @pl.when(c == 1)
        def _core1():
            run_edges(xsb_hbm)

        plsc.subcore_barrier()
        pltpu.sync_copy(accs.at[sl], out_hbm.at[c, sl])

    return prop_kernel


def _tc1_body(x_ref, w_ref, deg_ref, xsa_ref, xsb_ref, dinv_ref):
    n = x_ref.shape[0]
    npad = deg_ref.shape[1]
    ones = jnp.ones((deg_ref.shape[0], 1), jnp.float32)
    deg_col = lax.dot_general(deg_ref[...], ones, (((0,), (0,)), ((), ())),
                              preferred_element_type=jnp.float32)
    dinv = lax.rsqrt(deg_col + 1.0)
    xw = jnp.dot(x_ref[...], w_ref[...], preferred_element_type=jnp.float32)
    xs = (xw * dinv[:n]).astype(jnp.bfloat16)
    hh = xs.shape[1] // 2
    zpad = jnp.zeros((npad - n, hh), jnp.bfloat16)
    xsa_ref[pl.ds(0, n)] = xs[:, :hh]
    xsa_ref[pl.ds(n, npad - n)] = zpad
    xsb_ref[pl.ds(0, n)] = xs[:, hh:]
    xsb_ref[pl.ds(n, npad - n)] = zpad
    dinv_ref[...] = dinv


def _tc2_body(acc_ref, xsa_ref, xsb_ref, dinv_ref, w2a_ref, w2b_ref, b1_ref,
              outa_ref, outb_ref):
    dinv = dinv_ref[...]
    hh = acc_ref.shape[2]
    s0 = acc_ref[0].astype(jnp.float32) + xsa_ref[...].astype(jnp.float32)
    s1 = acc_ref[1].astype(jnp.float32) + xsb_ref[...].astype(jnp.float32)
    h1a = jnp.maximum(dinv * s0 + b1_ref[:, :hh], 0.0)
    h1b = jnp.maximum(dinv * s1 + b1_ref[:, hh:], 0.0)
    xw2 = (jnp.dot(h1a, w2a_ref[...], preferred_element_type=jnp.float32)
           + jnp.dot(h1b, w2b_ref[...], preferred_element_type=jnp.float32))
    xs2 = (xw2 * dinv).astype(jnp.bfloat16)
    outa_ref[...] = xs2[:, :hh]
    outb_ref[...] = xs2[:, hh:]


def _tc3_body(num_groups, acc_ref, xsa_ref, xsb_ref, dinv_ref, b2_ref,
              batch_ref, wpa_ref, wpb_ref, bp_ref, out_ref):
    dinv = dinv_ref[...]
    hh = acc_ref.shape[2]
    s0 = acc_ref[0].astype(jnp.float32) + xsa_ref[...].astype(jnp.float32)
    s1 = acc_ref[1].astype(jnp.float32) + xsb_ref[...].astype(jnp.float32)
    h2a = jnp.maximum(dinv * s0 + b2_ref[:, :hh], 0.0)
    h2b = jnp.maximum(dinv * s1 + b2_ref[:, hh:], 0.0)
    g = lax.broadcasted_iota(jnp.int32, (1, num_groups), 1)
    onehot = (batch_ref[...] == g).astype(jnp.float32)
    dims = (((0,), (0,)), ((), ()))
    sums_a = lax.dot_general(onehot, h2a, dims,
                             preferred_element_type=jnp.float32)
    sums_b = lax.dot_general(onehot, h2b, dims,
                             preferred_element_type=jnp.float32)
    ones = jnp.ones((h2a.shape[0], 1), jnp.float32)
    counts = lax.dot_general(onehot, ones, dims,
                             preferred_element_type=jnp.float32)
    inv_counts = 1.0 / jnp.maximum(counts, 1.0)
    pa = sums_a * inv_counts
    pb = sums_b * inv_counts
    out_ref[...] = (jnp.dot(pa, wpa_ref[...],
                            preferred_element_type=jnp.float32)
                    + jnp.dot(pb, wpb_ref[...],
                              preferred_element_type=jnp.float32)
                    + bp_ref[...])


def kernel(x, edge_index, batch, W1, b1, W2, b2, Wp, bp):
    n, d = x.shape
    e = edge_index.shape[1]
    h = W1.shape[1]
    hh = h // 2
    out_dim = Wp.shape[1]
    num_groups = 64

    chunk = 128
    npad = ((n + 2047) // 2048) * 2048
    ept = ((e + NS * chunk - 1) // (NS * chunk)) * chunk
    nchunks = ept // chunk

    edges4 = jnp.pad(edge_index, ((0, 0), (0, NS * ept - e)),
                     constant_values=n).reshape(2, NS, nchunks, chunk)
    batch_col = jnp.concatenate(
        [batch, jnp.full((npad - n,), num_groups, jnp.int32)]).reshape(npad, 1)
    zdeg = jnp.zeros((npad,), jnp.float32)
    zrows = jnp.zeros((npad // NS, hh), jnp.bfloat16)

    deg32 = _deg_call(npad, nchunks, chunk)(edges4, zdeg).reshape(
        NC * NS, npad)

    xs1a, xs1b, dinv = pl.pallas_call(
        _tc1_body,
        out_shape=[
            jax.ShapeDtypeStruct((npad, hh), jnp.bfloat16),
            jax.ShapeDtypeStruct((npad, hh), jnp.bfloat16),
            jax.ShapeDtypeStruct((npad, 1), jnp.float32),
        ],
    )(x, W1, deg32)

    prop = _prop_call(npad, hh, nchunks, chunk)
    acc1 = prop(xs1a, xs1b, edges4, zrows)

    xs2a, xs2b = pl.pallas_call(
        _tc2_body,
        out_shape=[
            jax.ShapeDtypeStruct((npad, hh), jnp.bfloat16),
            jax.ShapeDtypeStruct((npad, hh), jnp.bfloat16),
        ],
    )(acc1, xs1a, xs1b, dinv, W2[:hh], W2[hh:], b1.reshape(1, h))

    acc2 = prop(xs2a, xs2b, edges4, zrows)

    out = pl.pallas_call(
        functools.partial(_tc3_body, num_groups),
        out_shape=jax.ShapeDtypeStruct((num_groups, out_dim), jnp.float32),
    )(acc2, xs2a, xs2b, dinv, b2.reshape(1, h), batch_col, Wp[:hh], Wp[hh:],
      bp.reshape(1, out_dim))
    return out
